# baseline (speedup 1.0000x reference)
import functools

import jax
import jax.numpy as jnp
from jax import lax
from jax.experimental import pallas as pl
from jax.experimental.pallas import tpu as pltpu
from jax.experimental.pallas import tpu_sc as plsc

_NC, _NS, _L = 2, 16, 16
_NW = _NC * _NS
_E = 64
_T = 4 * 4096
_TPW = _T // _NW
_GRP = _TPW // _L
_WORDS = _TPW * _E

_LN2 = 0.6931471805599453
_SQRT2 = 1.4142135623730951


def _vlog(x):
    bits = plsc.bitcast(x, jnp.int32)
    e = (bits >> 23) - 127
    m = plsc.bitcast((bits & 0x007FFFFF) | 0x3F800000, jnp.float32)
    big = m > _SQRT2
    m = jnp.where(big, m * 0.5, m)
    e = jnp.where(big, e + 1, e)
    z = (m - 1.0) / (m + 1.0)
    z2 = z * z
    p = z * (2.0 + z2 * (2.0 / 3.0 + z2 * (2.0 / 5.0 + z2 * (2.0 / 7.0
        + z2 * (2.0 / 9.0)))))
    return e.astype(jnp.float32) * _LN2 + p


def _sc_body(t_hbm, s_hbm, out_hbm, t_v, s_v, acc_v, sem_t, sem_s):
    wid = lax.axis_index("s") * _NC + lax.axis_index("c")
    tok0 = wid * _TPW
    cp_t = pltpu.async_copy(t_hbm.at[pl.ds(tok0, _TPW)], t_v, sem_t)
    cp_s = pltpu.async_copy(s_hbm.at[pl.ds(tok0, _TPW)], s_v, sem_s)
    lane = lax.iota(jnp.int32, 16)
    def maxpass(ref, tok):
        m = [jnp.full((_L,), -jnp.inf, jnp.float32) for _ in range(4)]
        for e in range(_E):
            m[e % 4] = jnp.maximum(m[e % 4], plsc.load_gather(ref, [tok, lane ^ e]))
        return jnp.maximum(jnp.maximum(m[0], m[1]), jnp.maximum(m[2], m[3]))

    def grp(g, acc):
        tok = lane + g * _L
        mt = maxpass(t_v, tok)
        ms = maxpass(s_v, tok)
        st = [jnp.zeros((_L,), jnp.float32) for _ in range(4)]
        ss = [jnp.zeros((_L,), jnp.float32) for _ in range(4)]
        dt = [jnp.zeros((_L,), jnp.float32) for _ in range(4)]
        for e in range(_E):
            k = e % 4
            ex = lane ^ e
            tv = plsc.load_gather(t_v, [tok, ex])
            sv = plsc.load_gather(s_v, [tok, ex])
            et = jnp.exp(tv - mt)
            st[k] = st[k] + et
            dt[k] = dt[k] + et * (tv - sv)
            ss[k] = ss[k] + jnp.exp(sv - ms)
        stt = (st[0] + st[1]) + (st[2] + st[3])
        sst = (ss[0] + ss[1]) + (ss[2] + ss[3])
        dtt = (dt[0] + dt[1]) + (dt[2] + dt[3])
        kl = dtt / stt + (ms + _vlog(sst)) - (mt + _vlog(stt))
        return acc + kl

    cp_t.wait()
    cp_s.wait()
    acc_v[...] = lax.fori_loop(0, _GRP, grp, jnp.zeros((_L,), jnp.float32))
    pltpu.sync_copy(acc_v, out_hbm.at[wid])


def _make_kl(interpret: bool = False):
    mesh = plsc.VectorSubcoreMesh(
        core_axis_name="c", subcore_axis_name="s",
        num_cores=_NC, num_subcores=_NS)
    return pl.kernel(
        _sc_body,
        out_type=jax.ShapeDtypeStruct((_NW, _L), jnp.float32),
        mesh=mesh,
        scratch_types=[
            pltpu.VMEM((_TPW, _E), jnp.float32),
            pltpu.VMEM((_TPW, _E), jnp.float32),
            pltpu.VMEM((_L,), jnp.float32),
            pltpu.SemaphoreType.DMA,
            pltpu.SemaphoreType.DMA,
        ],
        compiler_params=pltpu.CompilerParams(needs_layout_passes=False, use_tc_tiling_on_sc=False),
        interpret=interpret,
    )


_KL = _make_kl()


def kernel(teacher_gates, student_gates, teacher_hidden_states,
           student_hidden_states, teacher_model, student_model,
           input_ids, attention_mask):
    t2 = teacher_gates.reshape(-1, teacher_gates.shape[-1])
    s2 = student_gates.reshape(-1, student_gates.shape[-1])
    parts = _KL(t2, s2)
    return jnp.sum(parts) / teacher_gates.shape[0]


# speedup vs baseline: 1.9339x; 1.9339x over previous
import functools

import jax
import jax.numpy as jnp
from jax import lax
from jax.experimental import pallas as pl
from jax.experimental.pallas import tpu as pltpu
from jax.experimental.pallas import tpu_sc as plsc

_NC, _NS, _L = 2, 16, 16
_NW = _NC * _NS
_E = 64
_T = 4 * 4096
_TPW = _T // _NW
_GRP = _TPW // _L

_LN2 = 0.6931471805599453
_SQRT2 = 1.4142135623730951


def _vlog(x):
    bits = plsc.bitcast(x, jnp.int32)
    e = (bits >> 23) - 127
    m = plsc.bitcast((bits & 0x007FFFFF) | 0x3F800000, jnp.float32)
    big = m > _SQRT2
    m = jnp.where(big, m * 0.5, m)
    e = jnp.where(big, e + 1, e)
    z = (m - 1.0) / (m + 1.0)
    z2 = z * z
    p = z * (2.0 + z2 * (2.0 / 3.0 + z2 * (2.0 / 5.0 + z2 * (2.0 / 7.0
        + z2 * (2.0 / 9.0)))))
    return e.astype(jnp.float32) * _LN2 + p


def _sc_body(t_hbm, s_hbm, out_hbm, t_v, s_v, acc_v, sem_t, sem_s):
    wid = lax.axis_index("s") * _NC + lax.axis_index("c")
    b = wid // 8
    s0 = (wid % 8) * _TPW
    cps = []
    for r in range(8):
        er = pl.ds(r * 8, 8)
        cps.append(pltpu.async_copy(
            t_hbm.at[b, er, pl.ds(s0, _TPW)], t_v.at[er, :], sem_t))
        cps.append(pltpu.async_copy(
            s_hbm.at[b, er, pl.ds(s0, _TPW)], s_v.at[er, :], sem_s))

    def grp(g, acc):
        tok = pl.ds(pl.multiple_of(g * _L, _L), _L)
        mt = [jnp.full((_L,), -jnp.inf, jnp.float32) for _ in range(4)]
        ms = [jnp.full((_L,), -jnp.inf, jnp.float32) for _ in range(4)]
        for e in range(_E):
            k = e % 4
            mt[k] = jnp.maximum(mt[k], t_v[e, tok])
            ms[k] = jnp.maximum(ms[k], s_v[e, tok])
        mtt = jnp.maximum(jnp.maximum(mt[0], mt[1]), jnp.maximum(mt[2], mt[3]))
        mst = jnp.maximum(jnp.maximum(ms[0], ms[1]), jnp.maximum(ms[2], ms[3]))
        st = [jnp.zeros((_L,), jnp.float32) for _ in range(4)]
        ss = [jnp.zeros((_L,), jnp.float32) for _ in range(4)]
        dt = [jnp.zeros((_L,), jnp.float32) for _ in range(4)]
        for e in range(_E):
            k = e % 4
            tv = t_v[e, tok]
            sv = s_v[e, tok]
            et = jnp.exp(tv - mtt)
            st[k] = st[k] + et
            dt[k] = dt[k] + et * (tv - sv)
            ss[k] = ss[k] + jnp.exp(sv - mst)
        stt = (st[0] + st[1]) + (st[2] + st[3])
        sst = (ss[0] + ss[1]) + (ss[2] + ss[3])
        dtt = (dt[0] + dt[1]) + (dt[2] + dt[3])
        kl = dtt / stt + (mst + _vlog(sst)) - (mtt + _vlog(stt))
        return acc + kl

    for cp in cps:
        cp.wait()
    acc_v[...] = lax.fori_loop(0, _GRP, grp, jnp.zeros((_L,), jnp.float32))
    pltpu.sync_copy(acc_v, out_hbm.at[wid])


def _make_kl(interpret: bool = False):
    mesh = plsc.VectorSubcoreMesh(
        core_axis_name="c", subcore_axis_name="s",
        num_cores=_NC, num_subcores=_NS)
    return pl.kernel(
        _sc_body,
        out_type=jax.ShapeDtypeStruct((_NW, _L), jnp.float32),
        mesh=mesh,
        scratch_types=[
            pltpu.VMEM((_E, _TPW), jnp.float32),
            pltpu.VMEM((_E, _TPW), jnp.float32),
            pltpu.VMEM((_L,), jnp.float32),
            pltpu.SemaphoreType.DMA,
            pltpu.SemaphoreType.DMA,
        ],
        compiler_params=pltpu.CompilerParams(needs_layout_passes=False),
        interpret=interpret,
    )


_KL = _make_kl()


def kernel(teacher_gates, student_gates, teacher_hidden_states,
           student_hidden_states, teacher_model, student_model,
           input_ids, attention_mask):
    tt = jnp.transpose(teacher_gates, (0, 2, 1))
    st = jnp.transpose(student_gates, (0, 2, 1))
    parts = _KL(tt, st)
    return jnp.sum(parts) / teacher_gates.shape[0]


# speedup vs baseline: 1.9721x; 1.0197x over previous
import functools

import jax
import jax.numpy as jnp
from jax import lax
from jax.experimental import pallas as pl
from jax.experimental.pallas import tpu as pltpu
from jax.experimental.pallas import tpu_sc as plsc

_NC, _NS, _L = 2, 16, 16
_NW = _NC * _NS
_E = 64
_T = 4 * 4096
_TPW = _T // _NW
_GRP = _TPW // _L

_LN2 = 0.6931471805599453
_SQRT2 = 1.4142135623730951


def _vlog(x):
    bits = plsc.bitcast(x, jnp.int32)
    e = (bits >> 23) - 127
    m = plsc.bitcast((bits & 0x007FFFFF) | 0x3F800000, jnp.float32)
    big = m > _SQRT2
    m = jnp.where(big, m * 0.5, m)
    e = jnp.where(big, e + 1, e)
    z = (m - 1.0) / (m + 1.0)
    z2 = z * z
    p = z * (2.0 + z2 * (2.0 / 3.0 + z2 * (2.0 / 5.0 + z2 * (2.0 / 7.0
        + z2 * (2.0 / 9.0)))))
    return e.astype(jnp.float32) * _LN2 + p


def _sc_body(t_hbm, s_hbm, out_hbm, t_v, s_v, acc_v, sem_t, sem_s):
    wid = lax.axis_index("s") * _NC + lax.axis_index("c")
    b = wid // 8
    s0 = (wid % 8) * _TPW
    cps = [
        pltpu.async_copy(t_hbm.at[b, :, pl.ds(s0, _TPW)], t_v, sem_t),
        pltpu.async_copy(s_hbm.at[b, :, pl.ds(s0, _TPW)], s_v, sem_s),
    ]

    def grp(g, acc):
        tok = pl.ds(pl.multiple_of(g * _L, _L), _L)
        mt = [jnp.full((_L,), -jnp.inf, jnp.float32) for _ in range(4)]
        ms = [jnp.full((_L,), -jnp.inf, jnp.float32) for _ in range(4)]
        for e in range(_E):
            k = e % 4
            mt[k] = jnp.maximum(mt[k], t_v[e, tok])
            ms[k] = jnp.maximum(ms[k], s_v[e, tok])
        mtt = jnp.maximum(jnp.maximum(mt[0], mt[1]), jnp.maximum(mt[2], mt[3]))
        mst = jnp.maximum(jnp.maximum(ms[0], ms[1]), jnp.maximum(ms[2], ms[3]))
        st = [jnp.zeros((_L,), jnp.float32) for _ in range(4)]
        ss = [jnp.zeros((_L,), jnp.float32) for _ in range(4)]
        dt = [jnp.zeros((_L,), jnp.float32) for _ in range(4)]
        for e in range(_E):
            k = e % 4
            tv = t_v[e, tok]
            sv = s_v[e, tok]
            et = jnp.exp(tv - mtt)
            st[k] = st[k] + et
            dt[k] = dt[k] + et * (tv - sv)
            ss[k] = ss[k] + jnp.exp(sv - mst)
        stt = (st[0] + st[1]) + (st[2] + st[3])
        sst = (ss[0] + ss[1]) + (ss[2] + ss[3])
        dtt = (dt[0] + dt[1]) + (dt[2] + dt[3])
        kl = dtt / stt + (mst + _vlog(sst)) - (mtt + _vlog(stt))
        return acc + kl

    for cp in cps:
        cp.wait()
    acc_v[...] = lax.fori_loop(0, _GRP, grp, jnp.zeros((_L,), jnp.float32))
    pltpu.sync_copy(acc_v, out_hbm.at[wid])


def _make_kl(interpret: bool = False):
    mesh = plsc.VectorSubcoreMesh(
        core_axis_name="c", subcore_axis_name="s",
        num_cores=_NC, num_subcores=_NS)
    return pl.kernel(
        _sc_body,
        out_type=jax.ShapeDtypeStruct((_NW, _L), jnp.float32),
        mesh=mesh,
        scratch_types=[
            pltpu.VMEM((_E, _TPW), jnp.float32),
            pltpu.VMEM((_E, _TPW), jnp.float32),
            pltpu.VMEM((_L,), jnp.float32),
            pltpu.SemaphoreType.DMA,
            pltpu.SemaphoreType.DMA,
        ],
        compiler_params=pltpu.CompilerParams(needs_layout_passes=False),
        interpret=interpret,
    )


_KL = _make_kl()


def kernel(teacher_gates, student_gates, teacher_hidden_states,
           student_hidden_states, teacher_model, student_model,
           input_ids, attention_mask):
    tt = jnp.transpose(teacher_gates, (0, 2, 1))
    st = jnp.transpose(student_gates, (0, 2, 1))
    parts = _KL(tt, st)
    return jnp.sum(parts) / teacher_gates.shape[0]


# speedup vs baseline: 2.2772x; 1.1547x over previous
import functools

import jax
import jax.numpy as jnp
from jax import lax
from jax.experimental import pallas as pl
from jax.experimental.pallas import tpu as pltpu
from jax.experimental.pallas import tpu_sc as plsc

_NC, _NS, _L = 2, 16, 16
_NW = _NC * _NS
_E = 64
_B, _S = 4, 4096
_SCB = 2
_T = _SCB * _S
_TPW = _T // _NW
_GRP = _TPW // _L

_LN2 = 0.6931471805599453
_SQRT2 = 1.4142135623730951


def _vlog(x):
    bits = plsc.bitcast(x, jnp.int32)
    e = (bits >> 23) - 127
    m = plsc.bitcast((bits & 0x007FFFFF) | 0x3F800000, jnp.float32)
    big = m > _SQRT2
    m = jnp.where(big, m * 0.5, m)
    e = jnp.where(big, e + 1, e)
    z = (m - 1.0) / (m + 1.0)
    z2 = z * z
    p = z * (2.0 + z2 * (2.0 / 3.0 + z2 * (2.0 / 5.0 + z2 * (2.0 / 7.0
        + z2 * (2.0 / 9.0)))))
    return e.astype(jnp.float32) * _LN2 + p


def _sc_body(t_hbm, s_hbm, out_hbm, t_v, s_v, acc_v, sem_t, sem_s):
    wid = lax.axis_index("s") * _NC + lax.axis_index("c")
    b = wid // 16
    s0 = (wid % 16) * _TPW
    cps = [
        pltpu.async_copy(t_hbm.at[b, :, pl.ds(s0, _TPW)], t_v, sem_t),
        pltpu.async_copy(s_hbm.at[b, :, pl.ds(s0, _TPW)], s_v, sem_s),
    ]

    def grp(g, acc):
        tok = pl.ds(pl.multiple_of(g * _L, _L), _L)
        mt = [jnp.full((_L,), -jnp.inf, jnp.float32) for _ in range(4)]
        ms = [jnp.full((_L,), -jnp.inf, jnp.float32) for _ in range(4)]
        for e in range(_E):
            k = e % 4
            mt[k] = jnp.maximum(mt[k], t_v[e, tok])
            ms[k] = jnp.maximum(ms[k], s_v[e, tok])
        mtt = jnp.maximum(jnp.maximum(mt[0], mt[1]), jnp.maximum(mt[2], mt[3]))
        mst = jnp.maximum(jnp.maximum(ms[0], ms[1]), jnp.maximum(ms[2], ms[3]))
        st = [jnp.zeros((_L,), jnp.float32) for _ in range(4)]
        ss = [jnp.zeros((_L,), jnp.float32) for _ in range(4)]
        dt = [jnp.zeros((_L,), jnp.float32) for _ in range(4)]
        for e in range(_E):
            k = e % 4
            tv = t_v[e, tok]
            sv = s_v[e, tok]
            et = jnp.exp(tv - mtt)
            st[k] = st[k] + et
            dt[k] = dt[k] + et * (tv - sv)
            ss[k] = ss[k] + jnp.exp(sv - mst)
        stt = (st[0] + st[1]) + (st[2] + st[3])
        sst = (ss[0] + ss[1]) + (ss[2] + ss[3])
        dtt = (dt[0] + dt[1]) + (dt[2] + dt[3])
        kl = dtt / stt + (mst + _vlog(sst)) - (mtt + _vlog(stt))
        return acc + kl

    for cp in cps:
        cp.wait()
    acc_v[...] = lax.fori_loop(0, _GRP, grp, jnp.zeros((_L,), jnp.float32))
    pltpu.sync_copy(acc_v, out_hbm.at[wid])


def _make_sc_kl(interpret: bool = False):
    mesh = plsc.VectorSubcoreMesh(
        core_axis_name="c", subcore_axis_name="s",
        num_cores=_NC, num_subcores=_NS)
    return pl.kernel(
        _sc_body,
        out_type=jax.ShapeDtypeStruct((_NW, _L), jnp.float32),
        mesh=mesh,
        scratch_types=[
            pltpu.VMEM((_E, _TPW), jnp.float32),
            pltpu.VMEM((_E, _TPW), jnp.float32),
            pltpu.VMEM((_L,), jnp.float32),
            pltpu.SemaphoreType.DMA,
            pltpu.SemaphoreType.DMA,
        ],
        compiler_params=pltpu.CompilerParams(needs_layout_passes=False),
        interpret=interpret,
    )


_SC_KL = _make_sc_kl()

_TCS = 512
_TC_GRID = (_B - _SCB) * (_S // _TCS)


def _tc_body(t_ref, s_ref, out_ref):
    i = pl.program_id(0)
    t = t_ref[0]
    s = s_ref[0]
    mt = jnp.max(t, axis=0)
    ms = jnp.max(s, axis=0)
    et = jnp.exp(t - mt[None, :])
    st = jnp.sum(et, axis=0)
    dt = jnp.sum(et * (t - s), axis=0)
    ss = jnp.sum(jnp.exp(s - ms[None, :]), axis=0)
    kl = dt / st + (ms + jnp.log(ss)) - (mt + jnp.log(st))

    @pl.when(i == 0)
    def _():
        out_ref[...] = jnp.zeros_like(out_ref)

    out_ref[...] += kl.reshape(4, 128)


_TC_KL = pl.pallas_call(
    _tc_body,
    grid=(_TC_GRID,),
    in_specs=[
        pl.BlockSpec((1, _E, _TCS),
                     lambda i: (_SCB + i // (_S // _TCS), 0, i % (_S // _TCS))),
        pl.BlockSpec((1, _E, _TCS),
                     lambda i: (_SCB + i // (_S // _TCS), 0, i % (_S // _TCS))),
    ],
    out_specs=pl.BlockSpec((4, 128), lambda i: (0, 0)),
    out_shape=jax.ShapeDtypeStruct((4, 128), jnp.float32),
)


def kernel(teacher_gates, student_gates, teacher_hidden_states,
           student_hidden_states, teacher_model, student_model,
           input_ids, attention_mask):
    tt = jnp.transpose(teacher_gates, (0, 2, 1))
    st = jnp.transpose(student_gates, (0, 2, 1))
    sc_parts = _SC_KL(tt, st)
    tc_parts = _TC_KL(tt, st)
    return (jnp.sum(sc_parts) + jnp.sum(tc_parts)) / teacher_gates.shape[0]


# speedup vs baseline: 2.2821x; 1.0021x over previous
import functools

import jax
import jax.numpy as jnp
from jax import lax
from jax.experimental import pallas as pl
from jax.experimental.pallas import tpu as pltpu
from jax.experimental.pallas import tpu_sc as plsc

_NC, _NS, _L = 2, 16, 16
_NW = _NC * _NS
_E = 64
_B, _S = 4, 4096
_SCB = 2
_T = _SCB * _S
_TPW = _T // _NW
_GRP = _TPW // _L

_LN2 = 0.6931471805599453
_SQRT2 = 1.4142135623730951


def _vlog(x):
    bits = plsc.bitcast(x, jnp.int32)
    e = (bits >> 23) - 127
    m = plsc.bitcast((bits & 0x007FFFFF) | 0x3F800000, jnp.float32)
    big = m > _SQRT2
    m = jnp.where(big, m * 0.5, m)
    e = jnp.where(big, e + 1, e)
    z = (m - 1.0) / (m + 1.0)
    z2 = z * z
    p = z * (2.0 + z2 * (2.0 / 3.0 + z2 * (2.0 / 5.0 + z2 * (2.0 / 7.0
        + z2 * (2.0 / 9.0)))))
    return e.astype(jnp.float32) * _LN2 + p


def _sc_body(t_hbm, s_hbm, out_hbm, t_v, s_v, acc_v, sem_t, sem_s):
    wid = lax.axis_index("s") * _NC + lax.axis_index("c")
    b = wid // 16
    s0 = (wid % 16) * _TPW
    cps = [
        pltpu.async_copy(t_hbm.at[b, :, pl.ds(s0, _TPW)], t_v, sem_t),
        pltpu.async_copy(s_hbm.at[b, :, pl.ds(s0, _TPW)], s_v, sem_s),
    ]

    def grp(g, acc):
        tok = pl.ds(pl.multiple_of(g * _L, _L), _L)
        mt = [jnp.full((_L,), -jnp.inf, jnp.float32) for _ in range(4)]
        ms = [jnp.full((_L,), -jnp.inf, jnp.float32) for _ in range(4)]
        for e in range(_E):
            k = e % 4
            mt[k] = jnp.maximum(mt[k], t_v[e, tok])
            ms[k] = jnp.maximum(ms[k], s_v[e, tok])
        mtt = jnp.maximum(jnp.maximum(mt[0], mt[1]), jnp.maximum(mt[2], mt[3]))
        mst = jnp.maximum(jnp.maximum(ms[0], ms[1]), jnp.maximum(ms[2], ms[3]))
        st = [jnp.zeros((_L,), jnp.float32) for _ in range(4)]
        ss = [jnp.zeros((_L,), jnp.float32) for _ in range(4)]
        dt = [jnp.zeros((_L,), jnp.float32) for _ in range(4)]
        for e in range(_E):
            k = e % 4
            tv = t_v[e, tok]
            sv = s_v[e, tok]
            et = jnp.exp(tv - mtt)
            st[k] = st[k] + et
            dt[k] = dt[k] + et * (tv - sv)
            ss[k] = ss[k] + jnp.exp(sv - mst)
        stt = (st[0] + st[1]) + (st[2] + st[3])
        sst = (ss[0] + ss[1]) + (ss[2] + ss[3])
        dtt = (dt[0] + dt[1]) + (dt[2] + dt[3])
        kl = dtt / stt + (mst + _vlog(sst)) - (mtt + _vlog(stt))
        return acc + kl

    for cp in cps:
        cp.wait()
    acc_v[...] = lax.fori_loop(0, _GRP, grp, jnp.zeros((_L,), jnp.float32))
    pltpu.sync_copy(acc_v, out_hbm.at[wid])


def _make_sc_kl(interpret: bool = False):
    mesh = plsc.VectorSubcoreMesh(
        core_axis_name="c", subcore_axis_name="s",
        num_cores=_NC, num_subcores=_NS)
    return pl.kernel(
        _sc_body,
        out_type=jax.ShapeDtypeStruct((_NW, _L), jnp.float32),
        mesh=mesh,
        scratch_types=[
            pltpu.VMEM((_E, _TPW), jnp.float32),
            pltpu.VMEM((_E, _TPW), jnp.float32),
            pltpu.VMEM((_L,), jnp.float32),
            pltpu.SemaphoreType.DMA,
            pltpu.SemaphoreType.DMA,
        ],
        compiler_params=pltpu.CompilerParams(needs_layout_passes=False),
        interpret=interpret,
    )


_SC_KL = _make_sc_kl()

_TC_GRID = _B - _SCB


def _tc_body(t_ref, s_ref, out_ref):
    i = pl.program_id(0)
    t = t_ref[0]
    s = s_ref[0]
    mt = jnp.max(t, axis=0)
    ms = jnp.max(s, axis=0)
    et = jnp.exp(t - mt[None, :])
    st = jnp.sum(et, axis=0)
    dt = jnp.sum(et * (t - s), axis=0)
    ss = jnp.sum(jnp.exp(s - ms[None, :]), axis=0)
    kl = dt / st + (ms + jnp.log(ss)) - (mt + jnp.log(st))

    @pl.when(i == 0)
    def _():
        out_ref[...] = jnp.zeros_like(out_ref)

    out_ref[...] += kl.reshape(32, 128)


_TC_KL = pl.pallas_call(
    _tc_body,
    grid=(_TC_GRID,),
    in_specs=[
        pl.BlockSpec((1, _E, _S), lambda i: (_SCB + i, 0, 0)),
        pl.BlockSpec((1, _E, _S), lambda i: (_SCB + i, 0, 0)),
    ],
    out_specs=pl.BlockSpec((32, 128), lambda i: (0, 0)),
    out_shape=jax.ShapeDtypeStruct((32, 128), jnp.float32),
)


def kernel(teacher_gates, student_gates, teacher_hidden_states,
           student_hidden_states, teacher_model, student_model,
           input_ids, attention_mask):
    tt = jnp.transpose(teacher_gates, (0, 2, 1))
    st = jnp.transpose(student_gates, (0, 2, 1))
    sc_parts = _SC_KL(tt, st)
    tc_parts = _TC_KL(tt, st)
    allp = jnp.concatenate([sc_parts.reshape(-1), tc_parts.reshape(-1)])
    return jnp.sum(allp) / teacher_gates.shape[0]


# speedup vs baseline: 2.5349x; 1.1108x over previous
import functools

import jax
import jax.numpy as jnp
from jax import lax
from jax.experimental import pallas as pl
from jax.experimental.pallas import tpu as pltpu
from jax.experimental.pallas import tpu_sc as plsc

_NC, _NS, _L = 2, 16, 16
_NW = _NC * _NS
_E = 64
_B, _S = 4, 4096
_SCB = 1
_T = _SCB * _S
_TPW = _T // _NW
_GRP = _TPW // _L

_LN2 = 0.6931471805599453
_SQRT2 = 1.4142135623730951


def _vlog(x):
    bits = plsc.bitcast(x, jnp.int32)
    e = (bits >> 23) - 127
    m = plsc.bitcast((bits & 0x007FFFFF) | 0x3F800000, jnp.float32)
    big = m > _SQRT2
    m = jnp.where(big, m * 0.5, m)
    e = jnp.where(big, e + 1, e)
    z = (m - 1.0) / (m + 1.0)
    z2 = z * z
    p = z * (2.0 + z2 * (2.0 / 3.0 + z2 * (2.0 / 5.0 + z2 * (2.0 / 7.0
        + z2 * (2.0 / 9.0)))))
    return e.astype(jnp.float32) * _LN2 + p


def _sc_body(t_hbm, s_hbm, out_hbm, t_v, s_v, acc_v, sem_t, sem_s):
    wid = lax.axis_index("s") * _NC + lax.axis_index("c")
    wpb = _NW // _SCB
    b = wid // wpb
    s0 = (wid % wpb) * _TPW
    cps = [
        pltpu.async_copy(t_hbm.at[b, :, pl.ds(s0, _TPW)], t_v, sem_t),
        pltpu.async_copy(s_hbm.at[b, :, pl.ds(s0, _TPW)], s_v, sem_s),
    ]

    def grp(g, acc):
        tok = pl.ds(pl.multiple_of(g * _L, _L), _L)
        mt = [jnp.full((_L,), -jnp.inf, jnp.float32) for _ in range(4)]
        ms = [jnp.full((_L,), -jnp.inf, jnp.float32) for _ in range(4)]
        for e in range(_E):
            k = e % 4
            mt[k] = jnp.maximum(mt[k], t_v[e, tok])
            ms[k] = jnp.maximum(ms[k], s_v[e, tok])
        mtt = jnp.maximum(jnp.maximum(mt[0], mt[1]), jnp.maximum(mt[2], mt[3]))
        mst = jnp.maximum(jnp.maximum(ms[0], ms[1]), jnp.maximum(ms[2], ms[3]))
        st = [jnp.zeros((_L,), jnp.float32) for _ in range(4)]
        ss = [jnp.zeros((_L,), jnp.float32) for _ in range(4)]
        dt = [jnp.zeros((_L,), jnp.float32) for _ in range(4)]
        for e in range(_E):
            k = e % 4
            tv = t_v[e, tok]
            sv = s_v[e, tok]
            et = jnp.exp(tv - mtt)
            st[k] = st[k] + et
            dt[k] = dt[k] + et * (tv - sv)
            ss[k] = ss[k] + jnp.exp(sv - mst)
        stt = (st[0] + st[1]) + (st[2] + st[3])
        sst = (ss[0] + ss[1]) + (ss[2] + ss[3])
        dtt = (dt[0] + dt[1]) + (dt[2] + dt[3])
        kl = dtt / stt + (mst + _vlog(sst)) - (mtt + _vlog(stt))
        return acc + kl

    for cp in cps:
        cp.wait()
    acc_v[...] = lax.fori_loop(0, _GRP, grp, jnp.zeros((_L,), jnp.float32))
    pltpu.sync_copy(acc_v, out_hbm.at[wid])


def _make_sc_kl(interpret: bool = False):
    mesh = plsc.VectorSubcoreMesh(
        core_axis_name="c", subcore_axis_name="s",
        num_cores=_NC, num_subcores=_NS)
    return pl.kernel(
        _sc_body,
        out_type=jax.ShapeDtypeStruct((_NW, _L), jnp.float32),
        mesh=mesh,
        scratch_types=[
            pltpu.VMEM((_E, _TPW), jnp.float32),
            pltpu.VMEM((_E, _TPW), jnp.float32),
            pltpu.VMEM((_L,), jnp.float32),
            pltpu.SemaphoreType.DMA,
            pltpu.SemaphoreType.DMA,
        ],
        compiler_params=pltpu.CompilerParams(needs_layout_passes=False),
        interpret=interpret,
    )


_SC_KL = _make_sc_kl()

_TC_GRID = _B - _SCB


def _tc_body(t_ref, s_ref, out_ref):
    i = pl.program_id(0)
    t = t_ref[0]
    s = s_ref[0]
    mt = jnp.max(t, axis=0)
    ms = jnp.max(s, axis=0)
    et = jnp.exp(t - mt[None, :])
    st = jnp.sum(et, axis=0)
    dt = jnp.sum(et * (t - s), axis=0)
    ss = jnp.sum(jnp.exp(s - ms[None, :]), axis=0)
    kl = dt / st + (ms + jnp.log(ss)) - (mt + jnp.log(st))

    @pl.when(i == 0)
    def _():
        out_ref[...] = jnp.zeros_like(out_ref)

    out_ref[...] += kl.reshape(32, 128)


_TC_KL = pl.pallas_call(
    _tc_body,
    grid=(_TC_GRID,),
    in_specs=[
        pl.BlockSpec((1, _E, _S), lambda i: (_SCB + i, 0, 0)),
        pl.BlockSpec((1, _E, _S), lambda i: (_SCB + i, 0, 0)),
    ],
    out_specs=pl.BlockSpec((32, 128), lambda i: (0, 0)),
    out_shape=jax.ShapeDtypeStruct((32, 128), jnp.float32),
)


def kernel(teacher_gates, student_gates, teacher_hidden_states,
           student_hidden_states, teacher_model, student_model,
           input_ids, attention_mask):
    tt = jnp.transpose(teacher_gates, (0, 2, 1))
    st = jnp.transpose(student_gates, (0, 2, 1))
    sc_parts = _SC_KL(tt, st)
    tc_parts = _TC_KL(tt, st)
    allp = jnp.concatenate([sc_parts.reshape(-1), tc_parts.reshape(-1)])
    return jnp.sum(allp) / teacher_gates.shape[0]


# speedup vs baseline: 2.5536x; 1.0074x over previous
import jax
import jax.numpy as jnp
from jax import lax
from jax.experimental import pallas as pl
from jax.experimental.pallas import tpu as pltpu
from jax.experimental.pallas import tpu_sc as plsc

_NC, _NS, _L = 2, 16, 16
_NW = _NC * _NS
_E = 64
_B, _S = 4, 4096
_SCB = 1
_T = _SCB * _S
_TPW = _T // _NW
_GRP = _TPW // _L

_LN2 = 0.6931471805599453
_SQRT2 = 1.4142135623730951


def _vlog(x):
    bits = plsc.bitcast(x, jnp.int32)
    e = (bits >> 23) - 127
    m = plsc.bitcast((bits & 0x007FFFFF) | 0x3F800000, jnp.float32)
    big = m > _SQRT2
    m = jnp.where(big, m * 0.5, m)
    e = jnp.where(big, e + 1, e)
    z = (m - 1.0) / (m + 1.0)
    z2 = z * z
    p = z * (2.0 + z2 * (2.0 / 3.0 + z2 * (2.0 / 5.0 + z2 * (2.0 / 7.0
        + z2 * (2.0 / 9.0)))))
    return e.astype(jnp.float32) * _LN2 + p


def _sc_body(t_hbm, s_hbm, out_hbm, t_v, s_v, acc_v, sem_t, sem_s):
    wid = lax.axis_index("s") * _NC + lax.axis_index("c")
    wpb = _NW // _SCB
    b = wid // wpb
    s0 = (wid % wpb) * _TPW
    cps = [
        pltpu.async_copy(t_hbm.at[b, :, pl.ds(s0, _TPW)], t_v, sem_t),
        pltpu.async_copy(s_hbm.at[b, :, pl.ds(s0, _TPW)], s_v, sem_s),
    ]

    def grp(g, acc):
        tok = pl.ds(pl.multiple_of(g * _L, _L), _L)
        mt = [jnp.full((_L,), -jnp.inf, jnp.float32) for _ in range(4)]
        ms = [jnp.full((_L,), -jnp.inf, jnp.float32) for _ in range(4)]
        for e in range(_E):
            k = e % 4
            mt[k] = jnp.maximum(mt[k], t_v[e, tok])
            ms[k] = jnp.maximum(ms[k], s_v[e, tok])
        mtt = jnp.maximum(jnp.maximum(mt[0], mt[1]), jnp.maximum(mt[2], mt[3]))
        mst = jnp.maximum(jnp.maximum(ms[0], ms[1]), jnp.maximum(ms[2], ms[3]))
        st = [jnp.zeros((_L,), jnp.float32) for _ in range(4)]
        ss = [jnp.zeros((_L,), jnp.float32) for _ in range(4)]
        dt = [jnp.zeros((_L,), jnp.float32) for _ in range(4)]
        for e in range(_E):
            k = e % 4
            tv = t_v[e, tok]
            sv = s_v[e, tok]
            et = jnp.exp(tv - mtt)
            st[k] = st[k] + et
            dt[k] = dt[k] + et * (tv - sv)
            ss[k] = ss[k] + jnp.exp(sv - mst)
        stt = (st[0] + st[1]) + (st[2] + st[3])
        sst = (ss[0] + ss[1]) + (ss[2] + ss[3])
        dtt = (dt[0] + dt[1]) + (dt[2] + dt[3])
        kl = dtt / stt + (mst + _vlog(sst)) - (mtt + _vlog(stt))
        return acc + kl

    for cp in cps:
        cp.wait()
    acc_v[...] = lax.fori_loop(0, _GRP, grp, jnp.zeros((_L,), jnp.float32))
    pltpu.sync_copy(acc_v, out_hbm.at[wid])


def _make_sc_kl(interpret: bool = False):
    mesh = plsc.VectorSubcoreMesh(
        core_axis_name="c", subcore_axis_name="s",
        num_cores=_NC, num_subcores=_NS)
    return pl.kernel(
        _sc_body,
        out_type=jax.ShapeDtypeStruct((_NW, _L), jnp.float32),
        mesh=mesh,
        scratch_types=[
            pltpu.VMEM((_E, _TPW), jnp.float32),
            pltpu.VMEM((_E, _TPW), jnp.float32),
            pltpu.VMEM((_L,), jnp.float32),
            pltpu.SemaphoreType.DMA,
            pltpu.SemaphoreType.DMA,
        ],
        compiler_params=pltpu.CompilerParams(needs_layout_passes=False),
        interpret=interpret,
    )


_SC_KL = _make_sc_kl()

_TC_GRID = _B - _SCB


def _tc_body(t_ref, s_ref, out_ref):
    i = pl.program_id(0)
    t = t_ref[0]
    s = s_ref[0]
    mt = jnp.max(t, axis=0)
    ms = jnp.max(s, axis=0)
    et = jnp.exp(t - mt[None, :])
    st = jnp.sum(et, axis=0)
    dt = jnp.sum(et * (t - s), axis=0)
    ss = jnp.sum(jnp.exp(s - ms[None, :]), axis=0)
    kl = dt / st + (ms + jnp.log(ss)) - (mt + jnp.log(st))

    @pl.when(i == 0)
    def _():
        out_ref[...] = jnp.zeros_like(out_ref)

    out_ref[...] += kl.reshape(32, 128)


_TC_KL = pl.pallas_call(
    _tc_body,
    grid=(_TC_GRID,),
    in_specs=[
        pl.BlockSpec((1, _E, _S), lambda i: (_SCB + i, 0, 0)),
        pl.BlockSpec((1, _E, _S), lambda i: (_SCB + i, 0, 0)),
    ],
    out_specs=pl.BlockSpec((32, 128), lambda i: (0, 0)),
    out_shape=jax.ShapeDtypeStruct((32, 128), jnp.float32),
)


def kernel(teacher_gates, student_gates, teacher_hidden_states,
           student_hidden_states, teacher_model, student_model,
           input_ids, attention_mask):
    tt = jnp.transpose(teacher_gates, (0, 2, 1))
    st = jnp.transpose(student_gates, (0, 2, 1))
    sc_parts = _SC_KL(tt, st)
    tc_parts = _TC_KL(tt, st)
    allp = jnp.concatenate([sc_parts.reshape(-1), tc_parts.reshape(-1)])
    return jnp.sum(allp) / teacher_gates.shape[0]
